# baseline (speedup 1.0000x reference)
import functools

import jax
import jax.numpy as jnp
from jax.experimental import pallas as pl

H = 128
STEPS = 16
N = 81
NP = 88


def _rrn_kernel(x_ref, pos_ref, inW_ref, inb_ref, g_ref, s_ref,
                w1sd_ref, b1_ref, w2_ref, b2_ref, w3_ref, b3_ref,
                w4_ref, b4_ref, n1ha_ref, n1x_ref, nb1_ref,
                n2_ref, nb2_ref, n3_ref, nb3_ref, n4_ref, nb4_ref,
                lng_ref, lnb_ref, outW_ref, outb_ref, out_ref, *, tb):
    f32 = jnp.float32
    bf16 = jnp.bfloat16
    dot = functools.partial(jnp.dot, preferred_element_type=f32)

    rows = tb * NP
    x = x_ref[...].reshape(rows, 10)
    pos = jnp.broadcast_to(pos_ref[...][None], (tb, NP, H)).reshape(rows, H)
    xe = dot(x, inW_ref[...]) + inb_ref[...] + pos

    g = g_ref[...]
    s = s_ref[...]
    w1sd = w1sd_ref[...]
    b1 = b1_ref[...]
    w2 = w2_ref[...]
    b2 = b2_ref[...]
    w3 = w3_ref[...]
    b3 = b3_ref[...]
    w4 = w4_ref[...]
    b4 = b4_ref[...]
    n1ha = n1ha_ref[...]
    nb1 = nb1_ref[...]
    n2 = n2_ref[...]
    nb2 = nb2_ref[...]
    n3 = n3_ref[...]
    nb3 = nb3_ref[...]
    n4 = n4_ref[...]
    nb4 = nb4_ref[...]
    lng = lng_ref[...]
    lnb = lnb_ref[...]

    xe_proj = dot(xe, n1x_ref[...]) + nb1
    pairs = tb // 2
    half = pairs * NP

    def step(_, h):
        ab = dot(h, w1sd)
        ms = []
        for p in range(pairs):
            r0 = ab[(2 * p) * NP:(2 * p + 1) * NP]
            r1 = ab[(2 * p + 1) * NP:(2 * p + 2) * NP]
            r = jnp.concatenate(
                [jnp.concatenate([r0[:, :H], r1[:, :H]], axis=1),
                 jnp.concatenate([r0[:, H:], r1[:, H:]], axis=1),
                 b1], axis=0)
            ms.append(jax.nn.relu(dot(g, r.astype(bf16))).astype(bf16))
        ms = [(jax.nn.relu(dot(m, w2) + b2)).astype(bf16) for m in ms]
        ms = [(jax.nn.relu(dot(m, w3) + b3)).astype(bf16) for m in ms]
        ms = [(dot(m, w4) + b4).astype(bf16) for m in ms]
        aggs = []
        for m in ms:
            a2 = dot(s, m)
            aggs.append(a2[:, :H])
            aggs.append(a2[:, H:])
        agg = jnp.concatenate(aggs, axis=0)
        u = jax.nn.relu(dot(jnp.concatenate([h, agg], axis=1), n1ha)
                        + xe_proj)
        u = jnp.concatenate([u[:half], u[half:]], axis=1)
        u = jax.nn.relu(dot(u, n2) + nb2)
        u = jax.nn.relu(dot(u, n3) + nb3)
        u = dot(u, n4) + nb4
        u = jnp.concatenate([u[:, :H], u[:, H:]], axis=0)
        h = h + u
        mu = jnp.mean(h, axis=1, keepdims=True)
        var = jnp.mean((h - mu) ** 2, axis=1, keepdims=True)
        h = (h - mu) / jnp.sqrt(var + 1e-5) * lng + lnb
        return h

    h = jax.lax.fori_loop(0, STEPS, step, xe)
    logits = dot(h, outW_ref[...]) + outb_ref[...]
    out_ref[...] = logits.reshape(tb, NP, H)


def _pair_block(w):
    z = jnp.zeros_like(w)
    return jnp.concatenate(
        [jnp.concatenate([w, z], axis=1),
         jnp.concatenate([z, w], axis=1)], axis=0)


def kernel(x, edge_index, params):
    B = x.shape[0]
    tb = 8
    src = edge_index[0]
    dst = edge_index[1]

    e = edge_index.shape[1]
    gcat = jnp.concatenate([jax.nn.one_hot(src, NP, dtype=jnp.float32),
                            jax.nn.one_hot(dst, NP, dtype=jnp.float32),
                            jnp.ones((e, 1), jnp.float32)],
                           axis=1).astype(jnp.bfloat16)
    scat = jax.nn.one_hot(dst, NP, dtype=jnp.bfloat16).T

    p = params
    xp = jnp.pad(x, ((0, 0), (0, NP - N), (0, 0)))
    pos_p = jnp.pad(p['pos'], ((0, NP - N), (0, 0)))
    msg = p['msg']
    node = p['node']
    w1 = msg[0][0]
    n1 = node[0][0]
    outW = jnp.pad(p['out_W'], ((0, 0), (0, H - 9)))
    outb = jnp.pad(p['out_b'], ((0, H - 9)))

    def row2(v):
        r = v.reshape(1, -1)
        return jnp.concatenate([r, r], axis=1)

    def row(v):
        return v.reshape(1, -1)

    w1sd = jnp.concatenate([w1[:H], w1[H:]], axis=1)
    n1ha = jnp.concatenate([n1[:H], n1[2 * H:]], axis=0)

    args = (
        xp, pos_p, p['in_W'], row(p['in_b']), gcat, scat,
        w1sd, row2(msg[0][1]),
        _pair_block(msg[1][0]).astype(jnp.bfloat16), row2(msg[1][1]),
        _pair_block(msg[2][0]).astype(jnp.bfloat16), row2(msg[2][1]),
        _pair_block(msg[3][0]).astype(jnp.bfloat16), row2(msg[3][1]),
        n1ha, n1[H:2 * H], row(node[0][1]),
        _pair_block(node[1][0]), row2(node[1][1]),
        _pair_block(node[2][0]), row2(node[2][1]),
        _pair_block(node[3][0]), row2(node[3][1]),
        row(p['ln_g']), row(p['ln_b']),
        outW, row(outb),
    )

    full = lambda a: pl.BlockSpec(a.shape, lambda i: (0,) * a.ndim)
    in_specs = [pl.BlockSpec((tb, NP, 10), lambda i: (i, 0, 0))]
    in_specs += [full(a) for a in args[1:]]

    out = pl.pallas_call(
        functools.partial(_rrn_kernel, tb=tb),
        grid=(B // tb,),
        in_specs=in_specs,
        out_specs=pl.BlockSpec((tb, NP, H), lambda i: (i, 0, 0)),
        out_shape=jax.ShapeDtypeStruct((B, NP, H), jnp.float32),
    )(*args)
    return out[:, :N, :9]


# speedup vs baseline: 1.0916x; 1.0916x over previous
import functools

import jax
import jax.numpy as jnp
from jax.experimental import pallas as pl

H = 128
STEPS = 16
N = 81
NP = 88


def _rrn_kernel(x_ref, pos_ref, inW_ref, inb_ref, g_ref, s_ref,
                w1sd_ref, b1_ref, w2_ref, b2_ref, w3_ref, b3_ref,
                w4_ref, b4_ref, n1ha_ref, n1x_ref, nb1_ref,
                n2_ref, nb2_ref, n3_ref, nb3_ref, n4_ref, nb4_ref,
                lng_ref, lnb_ref, outW_ref, outb_ref, out_ref, *, tb):
    f32 = jnp.float32
    bf16 = jnp.bfloat16
    dot = functools.partial(jnp.dot, preferred_element_type=f32)

    rows = tb * NP
    x = x_ref[...].reshape(rows, 10)
    pos = jnp.broadcast_to(pos_ref[...][None], (tb, NP, H)).reshape(rows, H)
    xe = dot(x, inW_ref[...]) + inb_ref[...] + pos

    g = g_ref[...]
    s = s_ref[...]
    w1sd = w1sd_ref[...]
    b1 = b1_ref[...]
    w2 = w2_ref[...]
    b2 = b2_ref[...]
    w3 = w3_ref[...]
    b3 = b3_ref[...]
    w4 = w4_ref[...]
    b4 = b4_ref[...]
    n1ha = n1ha_ref[...]
    nb1 = nb1_ref[...]
    n2 = n2_ref[...]
    nb2 = nb2_ref[...]
    n3 = n3_ref[...]
    nb3 = nb3_ref[...]
    n4 = n4_ref[...]
    nb4 = nb4_ref[...]
    lng = lng_ref[...]
    lnb = lnb_ref[...]

    xe_proj = dot(xe, n1x_ref[...]) + nb1
    pairs = tb // 2
    half = pairs * NP

    def step(_, h):
        ab = dot(h, w1sd)
        ms = []
        for p in range(pairs):
            r0 = ab[(2 * p) * NP:(2 * p + 1) * NP]
            r1 = ab[(2 * p + 1) * NP:(2 * p + 2) * NP]
            r = jnp.concatenate(
                [jnp.concatenate([r0[:, :H], r1[:, :H]], axis=1),
                 jnp.concatenate([r0[:, H:], r1[:, H:]], axis=1),
                 b1], axis=0)
            ms.append(jax.nn.relu(dot(g, r)))
        ms = [jax.nn.relu(dot(m, w2) + b2) for m in ms]
        ms = [jax.nn.relu(dot(m, w3) + b3) for m in ms]
        ms = [dot(m, w4) + b4 for m in ms]
        aggs = []
        for m in ms:
            a2 = dot(s, m)
            aggs.append(a2[:, :H])
            aggs.append(a2[:, H:])
        agg = jnp.concatenate(aggs, axis=0)
        u = jax.nn.relu(dot(jnp.concatenate([h, agg], axis=1), n1ha)
                        + xe_proj)
        u = jnp.concatenate([u[:half], u[half:]], axis=1)
        u = jax.nn.relu(dot(u, n2) + nb2)
        u = jax.nn.relu(dot(u, n3) + nb3)
        u = dot(u, n4) + nb4
        u = jnp.concatenate([u[:, :H], u[:, H:]], axis=0)
        h = h + u
        mu = jnp.mean(h, axis=1, keepdims=True)
        var = jnp.mean((h - mu) ** 2, axis=1, keepdims=True)
        h = (h - mu) / jnp.sqrt(var + 1e-5) * lng + lnb
        return h

    h = jax.lax.fori_loop(0, STEPS, step, xe)
    logits = dot(h, outW_ref[...]) + outb_ref[...]
    out_ref[...] = logits.reshape(tb, NP, H)


def _pair_block(w):
    z = jnp.zeros_like(w)
    return jnp.concatenate(
        [jnp.concatenate([w, z], axis=1),
         jnp.concatenate([z, w], axis=1)], axis=0)


def kernel(x, edge_index, params):
    B = x.shape[0]
    tb = 16
    src = edge_index[0]
    dst = edge_index[1]

    e = edge_index.shape[1]
    gcat = jnp.concatenate([jax.nn.one_hot(src, NP, dtype=jnp.float32),
                            jax.nn.one_hot(dst, NP, dtype=jnp.float32),
                            jnp.ones((e, 1), jnp.float32)],
                           axis=1)
    scat = jax.nn.one_hot(dst, NP, dtype=jnp.float32).T

    p = params
    xp = jnp.pad(x, ((0, 0), (0, NP - N), (0, 0)))
    pos_p = jnp.pad(p['pos'], ((0, NP - N), (0, 0)))
    msg = p['msg']
    node = p['node']
    w1 = msg[0][0]
    n1 = node[0][0]
    outW = jnp.pad(p['out_W'], ((0, 0), (0, H - 9)))
    outb = jnp.pad(p['out_b'], ((0, H - 9)))

    def row2(v):
        r = v.reshape(1, -1)
        return jnp.concatenate([r, r], axis=1)

    def row(v):
        return v.reshape(1, -1)

    w1sd = jnp.concatenate([w1[:H], w1[H:]], axis=1)
    n1ha = jnp.concatenate([n1[:H], n1[2 * H:]], axis=0)

    args = (
        xp, pos_p, p['in_W'], row(p['in_b']), gcat, scat,
        w1sd, row2(msg[0][1]),
        _pair_block(msg[1][0]), row2(msg[1][1]),
        _pair_block(msg[2][0]), row2(msg[2][1]),
        _pair_block(msg[3][0]), row2(msg[3][1]),
        n1ha, n1[H:2 * H], row(node[0][1]),
        _pair_block(node[1][0]), row2(node[1][1]),
        _pair_block(node[2][0]), row2(node[2][1]),
        _pair_block(node[3][0]), row2(node[3][1]),
        row(p['ln_g']), row(p['ln_b']),
        outW, row(outb),
    )

    full = lambda a: pl.BlockSpec(a.shape, lambda i: (0,) * a.ndim)
    in_specs = [pl.BlockSpec((tb, NP, 10), lambda i: (i, 0, 0))]
    in_specs += [full(a) for a in args[1:]]

    out = pl.pallas_call(
        functools.partial(_rrn_kernel, tb=tb),
        grid=(B // tb,),
        in_specs=in_specs,
        out_specs=pl.BlockSpec((tb, NP, H), lambda i: (i, 0, 0)),
        out_shape=jax.ShapeDtypeStruct((B, NP, H), jnp.float32),
    )(*args)
    return out[:, :N, :9]


# speedup vs baseline: 1.0931x; 1.0013x over previous
import functools

import jax
import jax.numpy as jnp
from jax.experimental import pallas as pl

H = 128
STEPS = 16
N = 81
NP = 88


def _rrn_kernel(x_ref, pos_ref, extra_ref, inW_ref, inb_ref, g_ref, s_ref,
                w1sd_ref, b1_ref, w2_ref, b2_ref, w3_ref, b3_ref,
                w4_ref, b4_ref, n1ha_ref, n1x_ref, nb1_ref,
                n2_ref, nb2_ref, n3_ref, nb3_ref, n4_ref, nb4_ref,
                lng_ref, lnb_ref, outW_ref, outb_ref, out_ref, *, tb):
    f32 = jnp.float32
    dot = functools.partial(jnp.dot, preferred_element_type=f32)

    rows = tb * NP
    x = x_ref[...].reshape(rows, 10)
    pos = jnp.broadcast_to(pos_ref[...][None], (tb, NP, H)).reshape(rows, H)
    xe = dot(x, inW_ref[...]) + inb_ref[...] + pos

    g = g_ref[...]
    s = s_ref[...]
    w1sd = w1sd_ref[...]
    b1 = b1_ref[...]
    w2 = w2_ref[...]
    b2 = b2_ref[...]
    w3 = w3_ref[...]
    b3 = b3_ref[...]
    w4 = w4_ref[...]
    b4 = b4_ref[...]
    n1ha = n1ha_ref[...]
    nb1 = nb1_ref[...]
    n2 = n2_ref[...]
    nb2 = nb2_ref[...]
    n3 = n3_ref[...]
    nb3 = nb3_ref[...]
    n4 = n4_ref[...]
    nb4 = nb4_ref[...]
    lng = lng_ref[...]
    lnb = lnb_ref[...]

    extra = jnp.broadcast_to(extra_ref[...][None], (tb, NP, H)).reshape(rows, H)
    xe_proj = dot(xe, n1x_ref[...]) + extra
    pairs = tb // 2
    half = pairs * NP

    def step(_, h):
        ab = dot(h, w1sd)
        ms = []
        for p in range(pairs):
            r0 = ab[(2 * p) * NP:(2 * p + 1) * NP]
            r1 = ab[(2 * p + 1) * NP:(2 * p + 2) * NP]
            r = jnp.concatenate(
                [jnp.concatenate([r0[:, :H], r1[:, :H]], axis=1),
                 jnp.concatenate([r0[:, H:], r1[:, H:]], axis=1),
                 b1], axis=0)
            ms.append(jax.nn.relu(dot(g, r)))
        ms = [jax.nn.relu(dot(m, w2) + b2) for m in ms]
        ms = [jax.nn.relu(dot(m, w3) + b3) for m in ms]
        ms = [dot(m, w4) for m in ms]
        aggs = []
        for m in ms:
            a2 = dot(s, m)
            aggs.append(a2[:, :H])
            aggs.append(a2[:, H:])
        agg = jnp.concatenate(aggs, axis=0)
        u = jax.nn.relu(dot(jnp.concatenate([h, agg], axis=1), n1ha)
                        + xe_proj)
        u = jnp.concatenate([u[:half], u[half:]], axis=1)
        u = jax.nn.relu(dot(u, n2) + nb2)
        u = jax.nn.relu(dot(u, n3) + nb3)
        u = dot(u, n4) + nb4
        u = jnp.concatenate([u[:, :H], u[:, H:]], axis=0)
        h = h + u
        mu = jnp.mean(h, axis=1, keepdims=True)
        var = jnp.mean((h - mu) ** 2, axis=1, keepdims=True)
        h = (h - mu) * jax.lax.rsqrt(var + 1e-5) * lng + lnb
        return h

    h = jax.lax.fori_loop(0, STEPS, step, xe)
    logits = dot(h, outW_ref[...]) + outb_ref[...]
    out_ref[...] = logits.reshape(tb, NP, H)


def _pair_block(w):
    z = jnp.zeros_like(w)
    return jnp.concatenate(
        [jnp.concatenate([w, z], axis=1),
         jnp.concatenate([z, w], axis=1)], axis=0)


def kernel(x, edge_index, params):
    B = x.shape[0]
    tb = 16
    src = edge_index[0]
    dst = edge_index[1]

    e = edge_index.shape[1]
    gcat = jnp.concatenate([jax.nn.one_hot(src, NP, dtype=jnp.float32),
                            jax.nn.one_hot(dst, NP, dtype=jnp.float32),
                            jnp.ones((e, 1), jnp.float32)],
                           axis=1)
    scat = jax.nn.one_hot(dst, NP, dtype=jnp.float32).T

    p = params
    xp = jnp.pad(x, ((0, 0), (0, NP - N), (0, 0)))
    pos_p = jnp.pad(p['pos'], ((0, NP - N), (0, 0)))
    msg = p['msg']
    node = p['node']
    w1 = msg[0][0]
    n1 = node[0][0]
    outW = jnp.pad(p['out_W'], ((0, 0), (0, H - 9)))
    outb = jnp.pad(p['out_b'], ((0, H - 9)))

    def row2(v):
        r = v.reshape(1, -1)
        return jnp.concatenate([r, r], axis=1)

    def row(v):
        return v.reshape(1, -1)

    w1sd = jnp.concatenate([w1[:H], w1[H:]], axis=1)
    n1ha = jnp.concatenate([n1[:H], n1[2 * H:]], axis=0)
    deg = jnp.sum(scat, axis=1)
    extra = (node[0][1].reshape(1, -1)
             + (deg[:, None] * msg[3][1][None, :]) @ n1[2 * H:])

    args = (
        xp, pos_p, extra, p['in_W'], row(p['in_b']), gcat, scat,
        w1sd, row2(msg[0][1]),
        _pair_block(msg[1][0]), row2(msg[1][1]),
        _pair_block(msg[2][0]), row2(msg[2][1]),
        _pair_block(msg[3][0]), row2(msg[3][1]),
        n1ha, n1[H:2 * H], row(node[0][1]),
        _pair_block(node[1][0]), row2(node[1][1]),
        _pair_block(node[2][0]), row2(node[2][1]),
        _pair_block(node[3][0]), row2(node[3][1]),
        row(p['ln_g']), row(p['ln_b']),
        outW, row(outb),
    )

    full = lambda a: pl.BlockSpec(a.shape, lambda i: (0,) * a.ndim)
    in_specs = [pl.BlockSpec((tb, NP, 10), lambda i: (i, 0, 0))]
    in_specs += [full(a) for a in args[1:]]

    out = pl.pallas_call(
        functools.partial(_rrn_kernel, tb=tb),
        grid=(B // tb,),
        in_specs=in_specs,
        out_specs=pl.BlockSpec((tb, NP, H), lambda i: (i, 0, 0)),
        out_shape=jax.ShapeDtypeStruct((B, NP, H), jnp.float32),
    )(*args)
    return out[:, :N, :9]


# speedup vs baseline: 1.1114x; 1.0168x over previous
import functools

import jax
import jax.numpy as jnp
from jax.experimental import pallas as pl

H = 128
STEPS = 16
N = 81
NP = 88


def _rrn_kernel(x_ref, pos_ref, extra_ref, inW_ref, inb_ref, g_ref, s_ref,
                w1sd_ref, b1_ref, w2_ref, b2_ref, w3_ref, b3_ref,
                w4_ref, b4_ref, n1ha_ref, n1x_ref, nb1_ref,
                n2_ref, nb2_ref, n3_ref, nb3_ref, n4_ref, nb4_ref,
                lng_ref, lnb_ref, outW_ref, outb_ref, out_ref, *, tb):
    f32 = jnp.float32
    dot = functools.partial(jnp.dot, preferred_element_type=f32)

    rows = tb * NP
    x = x_ref[...].reshape(rows, 10)
    pos = jnp.broadcast_to(pos_ref[...][None], (tb, NP, H)).reshape(rows, H)
    xe = dot(x, inW_ref[...]) + inb_ref[...] + pos

    g = g_ref[...]
    s = s_ref[...]
    w1sd = w1sd_ref[...]
    b1 = b1_ref[...]
    w2 = w2_ref[...]
    b2 = b2_ref[...]
    w3 = w3_ref[...]
    b3 = b3_ref[...]
    w4 = w4_ref[...]
    b4 = b4_ref[...]
    n1ha = n1ha_ref[...]
    nb1 = nb1_ref[...]
    n2 = n2_ref[...]
    nb2 = nb2_ref[...]
    n3 = n3_ref[...]
    nb3 = nb3_ref[...]
    n4 = n4_ref[...]
    nb4 = nb4_ref[...]
    lng = lng_ref[...]
    lnb = lnb_ref[...]

    extra = jnp.broadcast_to(extra_ref[...][None], (tb, NP, H)).reshape(rows, H)
    xe_proj = dot(xe, n1x_ref[...]) + extra
    pairs = tb // 2
    half = pairs * NP

    def step(_, h):
        ab = dot(h, w1sd)
        ms = []
        for p in range(pairs):
            r0 = ab[(2 * p) * NP:(2 * p + 1) * NP]
            r1 = ab[(2 * p + 1) * NP:(2 * p + 2) * NP]
            r = jnp.concatenate(
                [jnp.concatenate([r0[:, :H], r1[:, :H]], axis=1),
                 jnp.concatenate([r0[:, H:], r1[:, H:]], axis=1),
                 b1], axis=0)
            ms.append(jax.nn.relu(dot(g, r)))
        ms = [jax.nn.relu(dot(m, w2) + b2) for m in ms]
        ms = [jax.nn.relu(dot(m, w3) + b3) for m in ms]
        ms = [dot(m, w4) for m in ms]
        aggs = []
        for m in ms:
            a2 = dot(s, m)
            aggs.append(a2[:, :H])
            aggs.append(a2[:, H:])
        agg = jnp.concatenate(aggs, axis=0)
        u = jax.nn.relu(dot(jnp.concatenate([h, agg], axis=1), n1ha)
                        + xe_proj)
        u = jnp.concatenate([u[:half], u[half:]], axis=1)
        u = jax.nn.relu(dot(u, n2) + nb2)
        u = jax.nn.relu(dot(u, n3) + nb3)
        u = dot(u, n4) + nb4
        u = jnp.concatenate([u[:, :H], u[:, H:]], axis=0)
        h = h + u
        mu = jnp.mean(h, axis=1, keepdims=True)
        var = jnp.mean((h - mu) ** 2, axis=1, keepdims=True)
        h = (h - mu) * jax.lax.rsqrt(var + 1e-5) * lng + lnb
        return h

    h = jax.lax.fori_loop(0, STEPS, step, xe)
    logits = dot(h, outW_ref[...]) + outb_ref[...]
    out_ref[...] = logits.reshape(tb, NP, H)


def _pair_block(w):
    z = jnp.zeros_like(w)
    return jnp.concatenate(
        [jnp.concatenate([w, z], axis=1),
         jnp.concatenate([z, w], axis=1)], axis=0)


def kernel(x, edge_index, params):
    B = x.shape[0]
    tb = 32
    src = edge_index[0]
    dst = edge_index[1]

    e = edge_index.shape[1]
    gcat = jnp.concatenate([jax.nn.one_hot(src, NP, dtype=jnp.float32),
                            jax.nn.one_hot(dst, NP, dtype=jnp.float32),
                            jnp.ones((e, 1), jnp.float32)],
                           axis=1)
    scat = jax.nn.one_hot(dst, NP, dtype=jnp.float32).T

    p = params
    xp = jnp.pad(x, ((0, 0), (0, NP - N), (0, 0)))
    pos_p = jnp.pad(p['pos'], ((0, NP - N), (0, 0)))
    msg = p['msg']
    node = p['node']
    w1 = msg[0][0]
    n1 = node[0][0]
    outW = jnp.pad(p['out_W'], ((0, 0), (0, H - 9)))
    outb = jnp.pad(p['out_b'], ((0, H - 9)))

    def row2(v):
        r = v.reshape(1, -1)
        return jnp.concatenate([r, r], axis=1)

    def row(v):
        return v.reshape(1, -1)

    w1sd = jnp.concatenate([w1[:H], w1[H:]], axis=1)
    n1ha = jnp.concatenate([n1[:H], n1[2 * H:]], axis=0)
    deg = jnp.sum(scat, axis=1)
    extra = (node[0][1].reshape(1, -1)
             + (deg[:, None] * msg[3][1][None, :]) @ n1[2 * H:])

    args = (
        xp, pos_p, extra, p['in_W'], row(p['in_b']), gcat, scat,
        w1sd, row2(msg[0][1]),
        _pair_block(msg[1][0]), row2(msg[1][1]),
        _pair_block(msg[2][0]), row2(msg[2][1]),
        _pair_block(msg[3][0]), row2(msg[3][1]),
        n1ha, n1[H:2 * H], row(node[0][1]),
        _pair_block(node[1][0]), row2(node[1][1]),
        _pair_block(node[2][0]), row2(node[2][1]),
        _pair_block(node[3][0]), row2(node[3][1]),
        row(p['ln_g']), row(p['ln_b']),
        outW, row(outb),
    )

    full = lambda a: pl.BlockSpec(a.shape, lambda i: (0,) * a.ndim)
    in_specs = [pl.BlockSpec((tb, NP, 10), lambda i: (i, 0, 0))]
    in_specs += [full(a) for a in args[1:]]

    out = pl.pallas_call(
        functools.partial(_rrn_kernel, tb=tb),
        grid=(B // tb,),
        in_specs=in_specs,
        out_specs=pl.BlockSpec((tb, NP, H), lambda i: (i, 0, 0)),
        out_shape=jax.ShapeDtypeStruct((B, NP, H), jnp.float32),
    )(*args)
    return out[:, :N, :9]
